# baseline (speedup 1.0000x reference)
import functools

import jax
import jax.numpy as jnp
from jax import lax
from jax.experimental import pallas as pl
from jax.experimental.pallas import tpu as pltpu
from jax.experimental.pallas import tpu_sc as plsc

_NC = 2
_NS = 16
_B, _C, _H, _W, _K = 32, 8, 128, 128, 128
_HW = _H * _W


def _gather_body(outflat_hbm, index_hbm, out_hbm, idx_v, src_v, vals_v, sem):
    b = lax.axis_index("s") * _NC + lax.axis_index("c")

    pltpu.sync_copy(index_hbm.at[pl.ds(b * _K, _K)], idx_v)

    for c in range(_C):
        off = b * _C * _HW + c * _HW
        for j in range(8):
            src_v[pl.ds(c * _K + j * 16, 16)] = idx_v[pl.ds(j * 16, 16)] + off

    pltpu.async_copy(outflat_hbm.at[src_v], vals_v, sem).wait()

    pltpu.sync_copy(vals_v, out_hbm.at[pl.ds(b * _C * _K, _C * _K)])


@functools.partial(jax.jit)
def _sc_gather(outflat, index):
    mesh = plsc.VectorSubcoreMesh(core_axis_name="c", subcore_axis_name="s")
    kern = functools.partial(
        pl.kernel,
        mesh=mesh,
        out_type=jax.ShapeDtypeStruct((_B * _C * _K,), jnp.float32),
        scratch_types=[
            pltpu.VMEM((_K,), jnp.int32),
            pltpu.VMEM((_C * _K,), jnp.int32),
            pltpu.VMEM((_C * _K,), jnp.float32),
            pltpu.SemaphoreType.DMA,
        ],
    )(_gather_body)
    return kern(outflat, index)


def _loss_body(pred_ref, mask_ref, tb_ref, tr_ref, out_ref):
    m = mask_ref[...].astype(jnp.float32)
    o = [pred_ref[:, i, :] for i in range(8)]
    tb1 = tb_ref[0]
    tb2 = tb_ref[1]
    tr1 = tr_ref[0]
    tr2 = tr_ref[1]

    def ce_num(a, b, t):
        mx = jnp.maximum(a, b)
        logz = mx + jnp.log(jnp.exp(a - mx) + jnp.exp(b - mx))
        ll = jnp.where(t == 0, a, b)
        return jnp.sum((logz - ll) * m)

    msum = jnp.sum(m)
    bin_num = ce_num(o[0], o[1], tb1) + ce_num(o[4], o[5], tb2)
    loss_bin = jnp.where(msum > 0, bin_num / jnp.maximum(msum, 1.0), 0.0)

    def sl1(p, t):
        d = p - t
        ad = jnp.abs(d)
        return jnp.where(ad < 1.0, 0.5 * d * d, ad - 0.5)

    ind1 = (tb1 != 0).astype(jnp.float32)
    ind2 = (tb2 != 0).astype(jnp.float32)
    num1 = jnp.sum((sl1(o[2], jnp.sin(tr1)) + sl1(o[3], jnp.cos(tr1))) * ind1)
    num2 = jnp.sum((sl1(o[6], jnp.sin(tr2)) + sl1(o[7], jnp.cos(tr2))) * ind2)
    den1 = jnp.sum(ind1)
    den2 = jnp.sum(ind2)
    loss_res = jnp.where(den1 > 0, num1 / jnp.maximum(den1, 1.0), 0.0)
    loss_res += jnp.where(den2 > 0, num2 / jnp.maximum(den2, 1.0), 0.0)

    out_ref[0, 0] = loss_bin + loss_res


def _tc_loss(pred_bck, mask, tb, tr):
    return pl.pallas_call(
        _loss_body,
        out_shape=jax.ShapeDtypeStruct((1, 1), jnp.float32),
        out_specs=pl.BlockSpec(memory_space=pltpu.SMEM),
    )(pred_bck, mask, tb, tr)


def kernel(output, mask, index, rotbin, rotres):
    outflat = output.reshape(-1)
    predflat = _sc_gather(outflat, index.reshape(-1))
    pred_bck = predflat.reshape(_B, _C, _K)
    tb = rotbin.transpose(2, 0, 1)
    tr = rotres.transpose(2, 0, 1)
    loss = _tc_loss(pred_bck, mask, tb, tr)
    return loss[0, 0]


# speedup vs baseline: 1.0083x; 1.0083x over previous
import functools

import jax
import jax.numpy as jnp
from jax import lax
from jax.experimental import pallas as pl
from jax.experimental.pallas import tpu as pltpu
from jax.experimental.pallas import tpu_sc as plsc

_NC = 2
_NS = 16
_B, _C, _H, _W, _K = 32, 8, 128, 128, 128
_HW = _H * _W


def _gather_body(outflat_hbm, index_hbm, out_hbm, idx_v, vals_v, isem, gsem, osem):
    wid = lax.axis_index("s") * _NC + lax.axis_index("c")
    ch = wid // 4
    rb = wid % 4

    stage = [
        pltpu.async_copy(
            index_hbm.at[pl.ds(rb * 8 + h * 4, 4)],
            idx_v.at[pl.ds(h * 4, 4)],
            isem.at[h],
        )
        for h in range(2)
    ]

    gathers = []
    for h in range(2):
        stage[h].wait()
        for g in range(h * 4, h * 4 + 4):
            gathers.append(
                pltpu.async_copy(
                    outflat_hbm.at[pl.ds(((rb * 8 + g) * _C + ch) * _HW, _HW)].at[
                        idx_v.at[g]
                    ],
                    vals_v.at[g],
                    gsem.at[g],
                )
            )
    stores = []
    for g in range(8):
        gathers[g].wait()
        stores.append(
            pltpu.async_copy(
                vals_v.at[g],
                out_hbm.at[ch * _B + rb * 8 + g],
                osem.at[g],
            )
        )
    for s in stores:
        s.wait()


@functools.partial(jax.jit)
def _sc_gather(outflat, index):
    mesh = plsc.VectorSubcoreMesh(core_axis_name="c", subcore_axis_name="s")
    kern = functools.partial(
        pl.kernel,
        mesh=mesh,
        out_type=jax.ShapeDtypeStruct((_C * _B, _K), jnp.float32),
        scratch_types=[
            pltpu.VMEM((8, 128), jnp.int32),
            pltpu.VMEM((8, 128), jnp.float32),
            pltpu.SemaphoreType.DMA((2,)),
            pltpu.SemaphoreType.DMA((8,)),
            pltpu.SemaphoreType.DMA((8,)),
        ],
    )(_gather_body)
    return kern(outflat, index)


def _loss_body(pred_ref, mask_ref, tb_ref, tr_ref, out_ref):
    m = mask_ref[...].astype(jnp.float32)
    o = [pred_ref[i] for i in range(8)]
    tb1 = tb_ref[0]
    tb2 = tb_ref[1]
    tr1 = tr_ref[0]
    tr2 = tr_ref[1]

    def ce_num(a, b, t):
        mx = jnp.maximum(a, b)
        logz = mx + jnp.log(jnp.exp(a - mx) + jnp.exp(b - mx))
        ll = jnp.where(t == 0, a, b)
        return jnp.sum((logz - ll) * m)

    msum = jnp.sum(m)
    bin_num = ce_num(o[0], o[1], tb1) + ce_num(o[4], o[5], tb2)
    loss_bin = jnp.where(msum > 0, bin_num / jnp.maximum(msum, 1.0), 0.0)

    def sl1(p, t):
        d = p - t
        ad = jnp.abs(d)
        return jnp.where(ad < 1.0, 0.5 * d * d, ad - 0.5)

    ind1 = (tb1 != 0).astype(jnp.float32)
    ind2 = (tb2 != 0).astype(jnp.float32)
    num1 = jnp.sum((sl1(o[2], jnp.sin(tr1)) + sl1(o[3], jnp.cos(tr1))) * ind1)
    num2 = jnp.sum((sl1(o[6], jnp.sin(tr2)) + sl1(o[7], jnp.cos(tr2))) * ind2)
    den1 = jnp.sum(ind1)
    den2 = jnp.sum(ind2)
    loss_res = jnp.where(den1 > 0, num1 / jnp.maximum(den1, 1.0), 0.0)
    loss_res += jnp.where(den2 > 0, num2 / jnp.maximum(den2, 1.0), 0.0)

    out_ref[0, 0] = loss_bin + loss_res


def _tc_loss(pred_cm, mask, tb, tr):
    return pl.pallas_call(
        _loss_body,
        out_shape=jax.ShapeDtypeStruct((1, 1), jnp.float32),
        out_specs=pl.BlockSpec(memory_space=pltpu.SMEM),
    )(pred_cm, mask, tb, tr)


def kernel(output, mask, index, rotbin, rotres):
    outflat = output.reshape(-1)
    pred2d = _sc_gather(outflat, index)
    pred_cm = pred2d.reshape(_C, _B, _K)
    tb = rotbin.transpose(2, 0, 1)
    tr = rotres.transpose(2, 0, 1)
    loss = _tc_loss(pred_cm, mask, tb, tr)
    return loss[0, 0]
